# baseline (speedup 1.0000x reference)
import jax
import jax.numpy as jnp
from jax.experimental import pallas as pl
from jax.experimental.pallas import tpu as pltpu

_B = 64
_C = 4
_H = 224
_W = 224
_BB = 4
_NPIX = _B * _H * _W


def _loss_kernel(x_ref, y_ref, out_ref, acc_ref):
    step = pl.program_id(0)

    @pl.when(step == 0)
    def _init():
        acc_ref[0] = 0.0
        acc_ref[1] = 0.0
        acc_ref[2] = 0.0

    p = x_ref[:, 0]
    t = y_ref[:, 0]
    logp = jnp.maximum(jnp.log(p), -100.0)
    log1mp = jnp.maximum(jnp.log(1.0 - p), -100.0)
    nb = log1mp + t * (logp - log1mp)
    pt = jnp.exp(nb)
    one_m_pt = 1.0 - pt
    g = one_m_pt * one_m_pt * nb

    m = jnp.where(t != 0.0, 1.0, 0.0)

    d1 = y_ref[:, 1] - x_ref[:, 1]
    d2 = y_ref[:, 2] - x_ref[:, 2]
    d3 = y_ref[:, 3] - x_ref[:, 3]
    s = d1 * d1 + d2 * d2 + d3 * d3

    acc_ref[0] += jnp.sum(g)
    acc_ref[1] += jnp.sum(m * s)
    acc_ref[2] += jnp.sum(m)

    @pl.when(step == pl.num_programs(0) - 1)
    def _fini():
        out_ref[0] = -0.9 * acc_ref[0] / _NPIX + acc_ref[1] / (acc_ref[2] * 3.0)


def kernel(inputs, target):
    spec = pl.BlockSpec((_BB, _C, _H, _W), lambda b: (b, 0, 0, 0))
    out = pl.pallas_call(
        _loss_kernel,
        grid=(_B // _BB,),
        in_specs=[spec, spec],
        out_specs=pl.BlockSpec(memory_space=pltpu.SMEM),
        out_shape=jax.ShapeDtypeStruct((1,), jnp.float32),
        scratch_shapes=[pltpu.SMEM((3,), jnp.float32)],
    )(inputs, target)
    return out[0]
